# Initial kernel scaffold; baseline (speedup 1.0000x reference)
import jax
import jax.numpy as jnp
from jax.experimental import pallas as pl


def kernel(vertices, faces, face_probs):
    raise NotImplementedError("write your pallas kernel here")


# speedup vs baseline: 9.3808x; 9.3808x over previous
import functools

import jax
import jax.numpy as jnp
from jax.experimental import pallas as pl
from jax.experimental.pallas import tpu as pltpu

_TE = 128
_TM = 512
_TN = 512

_T1 = (1.0 + 1e-6) ** 2
_T2 = (1e-5 - 1e-8) ** 2


def _feature_body(vpad_ref, vpadT_ref, ac_ref, bc_ref, ar_ref, br_ref,
                  rowV_ref, rowC_ref, colV_ref, colC_ref, *, n_edges, V):
    t = pl.program_id(0)
    te = ac_ref.shape[0]

    lane_iota = jax.lax.broadcasted_iota(jnp.int32, (te, V), 1)
    oh_a = (lane_iota == ac_ref[...]).astype(jnp.float32)
    oh_b = (lane_iota == bc_ref[...]).astype(jnp.float32)
    pa = jnp.dot(oh_a, vpad_ref[...], preferred_element_type=jnp.float32)
    pb = jnp.dot(oh_b, vpad_ref[...], preferred_element_type=jnp.float32)
    v = pb - pa + 1e-8
    c = (pa + pb) * 0.5
    ge = t * te + jax.lax.broadcasted_iota(jnp.int32, (te, 1), 0)
    c = jnp.where(ge < n_edges, c, 1e6)
    rowV_ref[...] = v
    rowC_ref[...] = c

    sub_iota = jax.lax.broadcasted_iota(jnp.int32, (V, te), 0)
    oh_at = (sub_iota == ar_ref[...]).astype(jnp.float32)
    oh_bt = (sub_iota == br_ref[...]).astype(jnp.float32)
    pat = jnp.dot(vpadT_ref[...], oh_at, preferred_element_type=jnp.float32)
    pbt = jnp.dot(vpadT_ref[...], oh_bt, preferred_element_type=jnp.float32)
    vt = pbt - pat + 1e-8
    ct = (pat + pbt) * 0.5
    ger = t * te + jax.lax.broadcasted_iota(jnp.int32, (1, te), 1)
    ct = jnp.where(ger < n_edges, ct, 1e6)
    colV_ref[...] = vt
    colC_ref[...] = ct


def _pair_body(rowV_ref, rowC_ref, colV_ref, colC_ref, deg_ref):
    i = pl.program_id(0)
    j = pl.program_id(1)

    @pl.when((i == 0) & (j == 0))
    def _init():
        deg_ref[...] = jnp.zeros_like(deg_ref)

    vix = rowV_ref[:, 0:1]
    viy = rowV_ref[:, 1:2]
    viz = rowV_ref[:, 2:3]
    vjx = colV_ref[0:1, :]
    vjy = colV_ref[1:2, :]
    vjz = colV_ref[2:3, :]

    dx = rowC_ref[:, 0:1] - colC_ref[0:1, :]
    dy = rowC_ref[:, 1:2] - colC_ref[1:2, :]
    dz = rowC_ref[:, 2:3] - colC_ref[2:3, :]
    d2 = dx * dx + dy * dy + dz * dz

    c1 = viy * vjz - viz * vjy
    c2 = viz * vjx - vix * vjz
    c3 = vix * vjy - viy * vjx
    h = c1 * c1 + c2 * c2 + c3 * c3

    gi = i * _TM + jax.lax.broadcasted_iota(jnp.int32, (_TM, 1), 0)
    gj = j * _TN + jax.lax.broadcasted_iota(jnp.int32, (1, _TN), 1)
    m = (d2 < _T1) & (h > _T2) & (gi != gj)
    contrib = m.astype(jnp.float32)

    deg_ref[0:1, pl.ds(j * _TN, _TN)] += jnp.sum(contrib, axis=0, keepdims=True)


def _combine_body(x_ref, p_ref, out_ref, *, inv_f):
    cc = jnp.sum(x_ref[...], axis=1, keepdims=True)
    cc = jnp.clip(cc, 0.0, 100.0)
    out_ref[...] = jnp.sum(cc * p_ref[...], keepdims=True) * inv_f


def kernel(vertices, faces, face_probs):
    V = vertices.shape[0]
    F = faces.shape[0]
    E = 3 * F
    E_pad = ((E + _TM - 1) // _TM) * _TM
    F_pad = ((F + 7) // 8) * 8

    a = jnp.concatenate([faces[:, 0], faces[:, 1], faces[:, 2]])
    b = jnp.concatenate([faces[:, 1], faces[:, 2], faces[:, 0]])
    a = jnp.pad(a, (0, E_pad - E))
    b = jnp.pad(b, (0, E_pad - E))

    vpad = jnp.pad(vertices.astype(jnp.float32), ((0, 0), (0, 5)))
    vpadT = vpad.T

    feat = pl.pallas_call(
        functools.partial(_feature_body, n_edges=E, V=V),
        grid=(E_pad // _TE,),
        in_specs=[
            pl.BlockSpec((V, 8), lambda t: (0, 0)),
            pl.BlockSpec((8, V), lambda t: (0, 0)),
            pl.BlockSpec((_TE, 1), lambda t: (t, 0)),
            pl.BlockSpec((_TE, 1), lambda t: (t, 0)),
            pl.BlockSpec((1, _TE), lambda t: (0, t)),
            pl.BlockSpec((1, _TE), lambda t: (0, t)),
        ],
        out_specs=[
            pl.BlockSpec((_TE, 8), lambda t: (t, 0)),
            pl.BlockSpec((_TE, 8), lambda t: (t, 0)),
            pl.BlockSpec((8, _TE), lambda t: (0, t)),
            pl.BlockSpec((8, _TE), lambda t: (0, t)),
        ],
        out_shape=[
            jax.ShapeDtypeStruct((E_pad, 8), jnp.float32),
            jax.ShapeDtypeStruct((E_pad, 8), jnp.float32),
            jax.ShapeDtypeStruct((8, E_pad), jnp.float32),
            jax.ShapeDtypeStruct((8, E_pad), jnp.float32),
        ],
    )(vpad, vpadT, a.reshape(E_pad, 1), b.reshape(E_pad, 1),
      a.reshape(1, E_pad), b.reshape(1, E_pad))
    rowV, rowC, colV, colC = feat

    deg = pl.pallas_call(
        _pair_body,
        grid=(E_pad // _TM, E_pad // _TN),
        in_specs=[
            pl.BlockSpec((_TM, 8), lambda i, j: (i, 0)),
            pl.BlockSpec((_TM, 8), lambda i, j: (i, 0)),
            pl.BlockSpec((8, _TN), lambda i, j: (0, j)),
            pl.BlockSpec((8, _TN), lambda i, j: (0, j)),
        ],
        out_specs=pl.BlockSpec((8, E_pad), lambda i, j: (0, 0)),
        out_shape=jax.ShapeDtypeStruct((8, E_pad), jnp.float32),
    )(rowV, rowC, colV, colC)

    grouped = deg[0, :E].reshape(F, 3)
    grouped = jnp.pad(grouped, ((0, F_pad - F), (0, 0)))
    probs = jnp.pad(face_probs.astype(jnp.float32), (0, F_pad - F))

    out = pl.pallas_call(
        functools.partial(_combine_body, inv_f=1.0 / F),
        in_specs=[
            pl.BlockSpec((F_pad, 3), lambda: (0, 0)),
            pl.BlockSpec((F_pad, 1), lambda: (0, 0)),
        ],
        out_specs=pl.BlockSpec((1, 1), lambda: (0, 0)),
        out_shape=jax.ShapeDtypeStruct((1, 1), jnp.float32),
    )(grouped, probs.reshape(F_pad, 1))

    return out[0, 0]


# speedup vs baseline: 11.8002x; 1.2579x over previous
import functools

import jax
import jax.numpy as jnp
from jax.experimental import pallas as pl
from jax.experimental.pallas import tpu as pltpu

_TE = 128
_TM = 512
_TN = 512

_T1 = (1.0 + 1e-6) ** 2
_T2 = (1e-5 - 1e-8) ** 2


def _feature_body(vpad_ref, vpadT_ref, ac_ref, bc_ref, ar_ref, br_ref,
                  rowV_ref, rowC_ref, colV_ref, colC_ref, *, n_edges, V):
    t = pl.program_id(0)
    te = ac_ref.shape[0]

    lane_iota = jax.lax.broadcasted_iota(jnp.int32, (te, V), 1)
    oh_a = (lane_iota == ac_ref[...]).astype(jnp.float32)
    oh_b = (lane_iota == bc_ref[...]).astype(jnp.float32)
    pa = jnp.dot(oh_a, vpad_ref[...], preferred_element_type=jnp.float32)
    pb = jnp.dot(oh_b, vpad_ref[...], preferred_element_type=jnp.float32)
    v = pb - pa + 1e-8
    c = (pa + pb) * 0.5
    ge = t * te + jax.lax.broadcasted_iota(jnp.int32, (te, 1), 0)
    c = jnp.where(ge < n_edges, c, 1e6)
    rowV_ref[...] = v
    rowC_ref[...] = c

    sub_iota = jax.lax.broadcasted_iota(jnp.int32, (V, te), 0)
    oh_at = (sub_iota == ar_ref[...]).astype(jnp.float32)
    oh_bt = (sub_iota == br_ref[...]).astype(jnp.float32)
    pat = jnp.dot(vpadT_ref[...], oh_at, preferred_element_type=jnp.float32)
    pbt = jnp.dot(vpadT_ref[...], oh_bt, preferred_element_type=jnp.float32)
    vt = pbt - pat + 1e-8
    ct = (pat + pbt) * 0.5
    ger = t * te + jax.lax.broadcasted_iota(jnp.int32, (1, te), 1)
    ct = jnp.where(ger < n_edges, ct, 1e6)
    colV_ref[...] = vt
    colC_ref[...] = ct


def _pair_body(rowV_ref, rowC_ref, colV_ref, colC_ref, degc_ref, degr_ref):
    i = pl.program_id(0)
    j = pl.program_id(1)

    @pl.when((i == 0) & (j == 0))
    def _init():
        degc_ref[...] = jnp.zeros_like(degc_ref)
        degr_ref[...] = jnp.zeros_like(degr_ref)

    def predicate():
        vix = rowV_ref[:, 0:1]
        viy = rowV_ref[:, 1:2]
        viz = rowV_ref[:, 2:3]
        vjx = colV_ref[0:1, :]
        vjy = colV_ref[1:2, :]
        vjz = colV_ref[2:3, :]

        dx = rowC_ref[:, 0:1] - colC_ref[0:1, :]
        dy = rowC_ref[:, 1:2] - colC_ref[1:2, :]
        dz = rowC_ref[:, 2:3] - colC_ref[2:3, :]
        d2 = dx * dx + dy * dy + dz * dz

        c1 = viy * vjz - viz * vjy
        c2 = viz * vjx - vix * vjz
        c3 = vix * vjy - viy * vjx
        h = c1 * c1 + c2 * c2 + c3 * c3
        return (d2 < _T1) & (h > _T2)

    def accumulate(contrib):
        degc_ref[0:1, pl.ds(j * _TN, _TN)] += jnp.sum(
            contrib, axis=0, keepdims=True)
        degr_ref[pl.ds(i * _TM, _TM), 0:1] += jnp.sum(
            contrib, axis=1, keepdims=True)

    @pl.when(j > i)
    def _upper():
        accumulate(predicate().astype(jnp.float32))

    @pl.when(j == i)
    def _diag():
        li = jax.lax.broadcasted_iota(jnp.int32, (_TM, 1), 0)
        lj = jax.lax.broadcasted_iota(jnp.int32, (1, _TN), 1)
        accumulate((predicate() & (li < lj)).astype(jnp.float32))


def _combine_body(x_ref, y_ref, p_ref, out_ref, *, inv_f):
    cc = jnp.sum(x_ref[...] + y_ref[...], axis=1, keepdims=True)
    cc = jnp.clip(cc, 0.0, 100.0)
    out_ref[...] = jnp.sum(cc * p_ref[...], keepdims=True) * inv_f


def kernel(vertices, faces, face_probs):
    V = vertices.shape[0]
    F = faces.shape[0]
    E = 3 * F
    E_pad = ((E + _TM - 1) // _TM) * _TM
    F_pad = ((F + 7) // 8) * 8

    a = jnp.concatenate([faces[:, 0], faces[:, 1], faces[:, 2]])
    b = jnp.concatenate([faces[:, 1], faces[:, 2], faces[:, 0]])
    a = jnp.pad(a, (0, E_pad - E))
    b = jnp.pad(b, (0, E_pad - E))

    vpad = jnp.pad(vertices.astype(jnp.float32), ((0, 0), (0, 5)))
    vpadT = vpad.T

    feat = pl.pallas_call(
        functools.partial(_feature_body, n_edges=E, V=V),
        grid=(E_pad // _TE,),
        in_specs=[
            pl.BlockSpec((V, 8), lambda t: (0, 0)),
            pl.BlockSpec((8, V), lambda t: (0, 0)),
            pl.BlockSpec((_TE, 1), lambda t: (t, 0)),
            pl.BlockSpec((_TE, 1), lambda t: (t, 0)),
            pl.BlockSpec((1, _TE), lambda t: (0, t)),
            pl.BlockSpec((1, _TE), lambda t: (0, t)),
        ],
        out_specs=[
            pl.BlockSpec((_TE, 8), lambda t: (t, 0)),
            pl.BlockSpec((_TE, 8), lambda t: (t, 0)),
            pl.BlockSpec((8, _TE), lambda t: (0, t)),
            pl.BlockSpec((8, _TE), lambda t: (0, t)),
        ],
        out_shape=[
            jax.ShapeDtypeStruct((E_pad, 8), jnp.float32),
            jax.ShapeDtypeStruct((E_pad, 8), jnp.float32),
            jax.ShapeDtypeStruct((8, E_pad), jnp.float32),
            jax.ShapeDtypeStruct((8, E_pad), jnp.float32),
        ],
    )(vpad, vpadT, a.reshape(E_pad, 1), b.reshape(E_pad, 1),
      a.reshape(1, E_pad), b.reshape(1, E_pad))
    rowV, rowC, colV, colC = feat

    degc, degr = pl.pallas_call(
        _pair_body,
        grid=(E_pad // _TM, E_pad // _TN),
        in_specs=[
            pl.BlockSpec((_TM, 8), lambda i, j: (i, 0)),
            pl.BlockSpec((_TM, 8), lambda i, j: (i, 0)),
            pl.BlockSpec((8, _TN), lambda i, j: (0, j)),
            pl.BlockSpec((8, _TN), lambda i, j: (0, j)),
        ],
        out_specs=[
            pl.BlockSpec((8, E_pad), lambda i, j: (0, 0)),
            pl.BlockSpec((E_pad, 8), lambda i, j: (0, 0)),
        ],
        out_shape=[
            jax.ShapeDtypeStruct((8, E_pad), jnp.float32),
            jax.ShapeDtypeStruct((E_pad, 8), jnp.float32),
        ],
    )(rowV, rowC, colV, colC)

    grouped_c = jnp.pad(degc[0, :E].reshape(F, 3), ((0, F_pad - F), (0, 0)))
    grouped_r = jnp.pad(degr[:E, 0].reshape(F, 3), ((0, F_pad - F), (0, 0)))
    probs = jnp.pad(face_probs.astype(jnp.float32), (0, F_pad - F))

    out = pl.pallas_call(
        functools.partial(_combine_body, inv_f=1.0 / F),
        in_specs=[
            pl.BlockSpec((F_pad, 3), lambda: (0, 0)),
            pl.BlockSpec((F_pad, 3), lambda: (0, 0)),
            pl.BlockSpec((F_pad, 1), lambda: (0, 0)),
        ],
        out_specs=pl.BlockSpec((1, 1), lambda: (0, 0)),
        out_shape=jax.ShapeDtypeStruct((1, 1), jnp.float32),
    )(grouped_c, grouped_r, probs.reshape(F_pad, 1))

    return out[0, 0]


# speedup vs baseline: 11.8480x; 1.0041x over previous
import functools

import jax
import jax.numpy as jnp
from jax.experimental import pallas as pl
from jax.experimental.pallas import tpu as pltpu

_TE = 128
_TM = 512
_TN = 512

_T1 = (1.0 + 1e-6) ** 2
_T2 = (1e-5 - 1e-8) ** 2


def _feature_body(vpad_ref, vpadT_ref, ac_ref, bc_ref, ar_ref, br_ref,
                  rowV_ref, rowC_ref, colV_ref, colC_ref, *, n_edges, V):
    t = pl.program_id(0)
    te = ac_ref.shape[0]

    lane_iota = jax.lax.broadcasted_iota(jnp.int32, (te, V), 1)
    oh_a = (lane_iota == ac_ref[...]).astype(jnp.float32)
    oh_b = (lane_iota == bc_ref[...]).astype(jnp.float32)
    pa = jnp.dot(oh_a, vpad_ref[...], preferred_element_type=jnp.float32)
    pb = jnp.dot(oh_b, vpad_ref[...], preferred_element_type=jnp.float32)
    v = pb - pa + 1e-8
    c = (pa + pb) * 0.5
    ge = t * te + jax.lax.broadcasted_iota(jnp.int32, (te, 1), 0)
    c = jnp.where(ge < n_edges, c, 1e6)
    rowV_ref[...] = v
    rowC_ref[...] = c

    sub_iota = jax.lax.broadcasted_iota(jnp.int32, (V, te), 0)
    oh_at = (sub_iota == ar_ref[...]).astype(jnp.float32)
    oh_bt = (sub_iota == br_ref[...]).astype(jnp.float32)
    pat = jnp.dot(vpadT_ref[...], oh_at, preferred_element_type=jnp.float32)
    pbt = jnp.dot(vpadT_ref[...], oh_bt, preferred_element_type=jnp.float32)
    vt = pbt - pat + 1e-8
    ct = (pat + pbt) * 0.5
    ger = t * te + jax.lax.broadcasted_iota(jnp.int32, (1, te), 1)
    ct = jnp.where(ger < n_edges, ct, 1e6)
    colV_ref[...] = vt
    colC_ref[...] = ct


def _pair_body(rowV_ref, rowC_ref, colV_ref, colC_ref, degc_ref, degr_ref,
               racc_ref):
    i = pl.program_id(0)
    j = pl.program_id(1)
    ni = pl.num_programs(0)
    nj = pl.num_programs(1)

    @pl.when((i == 0) & (j == 0))
    def _init():
        degc_ref[...] = jnp.zeros_like(degc_ref)
        racc_ref[...] = jnp.zeros_like(racc_ref)

    def predicate():
        vix = rowV_ref[:, 0:1]
        viy = rowV_ref[:, 1:2]
        viz = rowV_ref[:, 2:3]
        vjx = colV_ref[0:1, :]
        vjy = colV_ref[1:2, :]
        vjz = colV_ref[2:3, :]

        dx = rowC_ref[:, 0:1] - colC_ref[0:1, :]
        dy = rowC_ref[:, 1:2] - colC_ref[1:2, :]
        dz = rowC_ref[:, 2:3] - colC_ref[2:3, :]
        d2 = dx * dx + dy * dy + dz * dz

        c1 = viy * vjz - viz * vjy
        c2 = viz * vjx - vix * vjz
        c3 = vix * vjy - viy * vjx
        h = c1 * c1 + c2 * c2 + c3 * c3
        return (d2 < _T1) & (h > _T2)

    def accumulate(contrib):
        degc_ref[0:1, pl.ds(j * _TN, _TN)] += jnp.sum(
            contrib, axis=0, keepdims=True)
        part = (contrib[:, 0:128] + contrib[:, 128:256]
                + contrib[:, 256:384] + contrib[:, 384:512])
        racc_ref[pl.ds(i * _TM, _TM), :] += part

    @pl.when(j > i)
    def _upper():
        accumulate(predicate().astype(jnp.float32))

    @pl.when(j == i)
    def _diag():
        li = jax.lax.broadcasted_iota(jnp.int32, (_TM, 1), 0)
        lj = jax.lax.broadcasted_iota(jnp.int32, (1, _TN), 1)
        accumulate((predicate() & (li < lj)).astype(jnp.float32))

    @pl.when((i == ni - 1) & (j == nj - 1))
    def _flush():
        degr_ref[...] = jnp.sum(racc_ref[...], axis=1, keepdims=True)


def _combine_body(x_ref, y_ref, p_ref, out_ref, *, inv_f):
    cc = jnp.sum(x_ref[...] + y_ref[...], axis=1, keepdims=True)
    cc = jnp.clip(cc, 0.0, 100.0)
    out_ref[...] = jnp.sum(cc * p_ref[...], keepdims=True) * inv_f


def kernel(vertices, faces, face_probs):
    V = vertices.shape[0]
    F = faces.shape[0]
    E = 3 * F
    E_pad = ((E + _TM - 1) // _TM) * _TM
    F_pad = ((F + 7) // 8) * 8

    a = jnp.concatenate([faces[:, 0], faces[:, 1], faces[:, 2]])
    b = jnp.concatenate([faces[:, 1], faces[:, 2], faces[:, 0]])
    a = jnp.pad(a, (0, E_pad - E))
    b = jnp.pad(b, (0, E_pad - E))

    vpad = jnp.pad(vertices.astype(jnp.float32), ((0, 0), (0, 5)))
    vpadT = vpad.T

    feat = pl.pallas_call(
        functools.partial(_feature_body, n_edges=E, V=V),
        grid=(E_pad // _TE,),
        in_specs=[
            pl.BlockSpec((V, 8), lambda t: (0, 0)),
            pl.BlockSpec((8, V), lambda t: (0, 0)),
            pl.BlockSpec((_TE, 1), lambda t: (t, 0)),
            pl.BlockSpec((_TE, 1), lambda t: (t, 0)),
            pl.BlockSpec((1, _TE), lambda t: (0, t)),
            pl.BlockSpec((1, _TE), lambda t: (0, t)),
        ],
        out_specs=[
            pl.BlockSpec((_TE, 8), lambda t: (t, 0)),
            pl.BlockSpec((_TE, 8), lambda t: (t, 0)),
            pl.BlockSpec((8, _TE), lambda t: (0, t)),
            pl.BlockSpec((8, _TE), lambda t: (0, t)),
        ],
        out_shape=[
            jax.ShapeDtypeStruct((E_pad, 8), jnp.float32),
            jax.ShapeDtypeStruct((E_pad, 8), jnp.float32),
            jax.ShapeDtypeStruct((8, E_pad), jnp.float32),
            jax.ShapeDtypeStruct((8, E_pad), jnp.float32),
        ],
    )(vpad, vpadT, a.reshape(E_pad, 1), b.reshape(E_pad, 1),
      a.reshape(1, E_pad), b.reshape(1, E_pad))
    rowV, rowC, colV, colC = feat

    degc, degr = pl.pallas_call(
        _pair_body,
        grid=(E_pad // _TM, E_pad // _TN),
        in_specs=[
            pl.BlockSpec((_TM, 8), lambda i, j: (i, 0)),
            pl.BlockSpec((_TM, 8), lambda i, j: (i, 0)),
            pl.BlockSpec((8, _TN), lambda i, j: (0, j)),
            pl.BlockSpec((8, _TN), lambda i, j: (0, j)),
        ],
        out_specs=[
            pl.BlockSpec((8, E_pad), lambda i, j: (0, 0)),
            pl.BlockSpec((E_pad, 1), lambda i, j: (0, 0)),
        ],
        out_shape=[
            jax.ShapeDtypeStruct((8, E_pad), jnp.float32),
            jax.ShapeDtypeStruct((E_pad, 1), jnp.float32),
        ],
        scratch_shapes=[pltpu.VMEM((E_pad, 128), jnp.float32)],
    )(rowV, rowC, colV, colC)

    grouped_c = jnp.pad(degc[0, :E].reshape(F, 3), ((0, F_pad - F), (0, 0)))
    grouped_r = jnp.pad(degr[:E, 0].reshape(F, 3), ((0, F_pad - F), (0, 0)))
    probs = jnp.pad(face_probs.astype(jnp.float32), (0, F_pad - F))

    out = pl.pallas_call(
        functools.partial(_combine_body, inv_f=1.0 / F),
        in_specs=[
            pl.BlockSpec((F_pad, 3), lambda: (0, 0)),
            pl.BlockSpec((F_pad, 3), lambda: (0, 0)),
            pl.BlockSpec((F_pad, 1), lambda: (0, 0)),
        ],
        out_specs=pl.BlockSpec((1, 1), lambda: (0, 0)),
        out_shape=jax.ShapeDtypeStruct((1, 1), jnp.float32),
    )(grouped_c, grouped_r, probs.reshape(F_pad, 1))

    return out[0, 0]


# speedup vs baseline: 14.1289x; 1.1925x over previous
import functools

import jax
import jax.numpy as jnp
from jax.experimental import pallas as pl
from jax.experimental.pallas import tpu as pltpu

_TE = 128
_TM = 1024
_TN = 1024

_T1 = (1.0 + 1e-6) ** 2
_T2 = (1e-5 - 1e-8) ** 2


def _feature_body(vpad_ref, vpadT_ref, ac_ref, bc_ref, ar_ref, br_ref,
                  rowV_ref, rowC_ref, colV_ref, colC_ref, *, n_edges, V):
    t = pl.program_id(0)
    te = ac_ref.shape[0]

    lane_iota = jax.lax.broadcasted_iota(jnp.int32, (te, V), 1)
    oh_a = (lane_iota == ac_ref[...]).astype(jnp.float32)
    oh_b = (lane_iota == bc_ref[...]).astype(jnp.float32)
    pa = jnp.dot(oh_a, vpad_ref[...], preferred_element_type=jnp.float32)
    pb = jnp.dot(oh_b, vpad_ref[...], preferred_element_type=jnp.float32)
    v = pb - pa + 1e-8
    c = (pa + pb) * 0.5
    ge = t * te + jax.lax.broadcasted_iota(jnp.int32, (te, 1), 0)
    c = jnp.where(ge < n_edges, c, 1e6)
    rowV_ref[...] = v
    rowC_ref[...] = c

    sub_iota = jax.lax.broadcasted_iota(jnp.int32, (V, te), 0)
    oh_at = (sub_iota == ar_ref[...]).astype(jnp.float32)
    oh_bt = (sub_iota == br_ref[...]).astype(jnp.float32)
    pat = jnp.dot(vpadT_ref[...], oh_at, preferred_element_type=jnp.float32)
    pbt = jnp.dot(vpadT_ref[...], oh_bt, preferred_element_type=jnp.float32)
    vt = pbt - pat + 1e-8
    ct = (pat + pbt) * 0.5
    ger = t * te + jax.lax.broadcasted_iota(jnp.int32, (1, te), 1)
    ct = jnp.where(ger < n_edges, ct, 1e6)
    colV_ref[...] = vt
    colC_ref[...] = ct


def _pair_body(rowV_ref, rowC_ref, colV_ref, colC_ref, degc_ref, degr_ref,
               racc_ref):
    i = pl.program_id(0)
    j = pl.program_id(1)
    ni = pl.num_programs(0)
    nj = pl.num_programs(1)

    @pl.when((i == 0) & (j == 0))
    def _init():
        degc_ref[...] = jnp.zeros_like(degc_ref)
        racc_ref[...] = jnp.zeros_like(racc_ref)

    def predicate():
        vix = rowV_ref[:, 0:1]
        viy = rowV_ref[:, 1:2]
        viz = rowV_ref[:, 2:3]
        vjx = colV_ref[0:1, :]
        vjy = colV_ref[1:2, :]
        vjz = colV_ref[2:3, :]

        dx = rowC_ref[:, 0:1] - colC_ref[0:1, :]
        dy = rowC_ref[:, 1:2] - colC_ref[1:2, :]
        dz = rowC_ref[:, 2:3] - colC_ref[2:3, :]
        d2 = dx * dx + dy * dy + dz * dz

        c1 = viy * vjz - viz * vjy
        c2 = viz * vjx - vix * vjz
        c3 = vix * vjy - viy * vjx
        h = c1 * c1 + c2 * c2 + c3 * c3
        return (d2 < _T1) & (h > _T2)

    def accumulate(contrib):
        degc_ref[0:1, pl.ds(j * _TN, _TN)] += jnp.sum(
            contrib, axis=0, keepdims=True)
        part = sum(contrib[:, k * 128:(k + 1) * 128] for k in range(1, _TN // 128)) + contrib[:, 0:128]
        racc_ref[pl.ds(i * _TM, _TM), :] += part

    @pl.when(j > i)
    def _upper():
        accumulate(predicate().astype(jnp.float32))

    @pl.when(j == i)
    def _diag():
        li = jax.lax.broadcasted_iota(jnp.int32, (_TM, 1), 0)
        lj = jax.lax.broadcasted_iota(jnp.int32, (1, _TN), 1)
        accumulate((predicate() & (li < lj)).astype(jnp.float32))

    @pl.when((i == ni - 1) & (j == nj - 1))
    def _flush():
        degr_ref[...] = jnp.sum(racc_ref[...], axis=1, keepdims=True)


def _combine_body(x_ref, y_ref, p_ref, out_ref, *, inv_f):
    cc = jnp.sum(x_ref[...] + y_ref[...], axis=1, keepdims=True)
    cc = jnp.clip(cc, 0.0, 100.0)
    out_ref[...] = jnp.sum(cc * p_ref[...], keepdims=True) * inv_f


def kernel(vertices, faces, face_probs):
    V = vertices.shape[0]
    F = faces.shape[0]
    E = 3 * F
    E_pad = ((E + _TM - 1) // _TM) * _TM
    F_pad = ((F + 7) // 8) * 8

    a = jnp.concatenate([faces[:, 0], faces[:, 1], faces[:, 2]])
    b = jnp.concatenate([faces[:, 1], faces[:, 2], faces[:, 0]])
    a = jnp.pad(a, (0, E_pad - E))
    b = jnp.pad(b, (0, E_pad - E))

    vpad = jnp.pad(vertices.astype(jnp.float32), ((0, 0), (0, 5)))
    vpadT = vpad.T

    feat = pl.pallas_call(
        functools.partial(_feature_body, n_edges=E, V=V),
        grid=(E_pad // _TE,),
        in_specs=[
            pl.BlockSpec((V, 8), lambda t: (0, 0)),
            pl.BlockSpec((8, V), lambda t: (0, 0)),
            pl.BlockSpec((_TE, 1), lambda t: (t, 0)),
            pl.BlockSpec((_TE, 1), lambda t: (t, 0)),
            pl.BlockSpec((1, _TE), lambda t: (0, t)),
            pl.BlockSpec((1, _TE), lambda t: (0, t)),
        ],
        out_specs=[
            pl.BlockSpec((_TE, 8), lambda t: (t, 0)),
            pl.BlockSpec((_TE, 8), lambda t: (t, 0)),
            pl.BlockSpec((8, _TE), lambda t: (0, t)),
            pl.BlockSpec((8, _TE), lambda t: (0, t)),
        ],
        out_shape=[
            jax.ShapeDtypeStruct((E_pad, 8), jnp.float32),
            jax.ShapeDtypeStruct((E_pad, 8), jnp.float32),
            jax.ShapeDtypeStruct((8, E_pad), jnp.float32),
            jax.ShapeDtypeStruct((8, E_pad), jnp.float32),
        ],
    )(vpad, vpadT, a.reshape(E_pad, 1), b.reshape(E_pad, 1),
      a.reshape(1, E_pad), b.reshape(1, E_pad))
    rowV, rowC, colV, colC = feat

    degc, degr = pl.pallas_call(
        _pair_body,
        grid=(E_pad // _TM, E_pad // _TN),
        in_specs=[
            pl.BlockSpec((_TM, 8), lambda i, j: (i, 0)),
            pl.BlockSpec((_TM, 8), lambda i, j: (i, 0)),
            pl.BlockSpec((8, _TN), lambda i, j: (0, j)),
            pl.BlockSpec((8, _TN), lambda i, j: (0, j)),
        ],
        out_specs=[
            pl.BlockSpec((8, E_pad), lambda i, j: (0, 0)),
            pl.BlockSpec((E_pad, 1), lambda i, j: (0, 0)),
        ],
        out_shape=[
            jax.ShapeDtypeStruct((8, E_pad), jnp.float32),
            jax.ShapeDtypeStruct((E_pad, 1), jnp.float32),
        ],
        scratch_shapes=[pltpu.VMEM((E_pad, 128), jnp.float32)],
    )(rowV, rowC, colV, colC)

    grouped_c = jnp.pad(degc[0, :E].reshape(F, 3), ((0, F_pad - F), (0, 0)))
    grouped_r = jnp.pad(degr[:E, 0].reshape(F, 3), ((0, F_pad - F), (0, 0)))
    probs = jnp.pad(face_probs.astype(jnp.float32), (0, F_pad - F))

    out = pl.pallas_call(
        functools.partial(_combine_body, inv_f=1.0 / F),
        in_specs=[
            pl.BlockSpec((F_pad, 3), lambda: (0, 0)),
            pl.BlockSpec((F_pad, 3), lambda: (0, 0)),
            pl.BlockSpec((F_pad, 1), lambda: (0, 0)),
        ],
        out_specs=pl.BlockSpec((1, 1), lambda: (0, 0)),
        out_shape=jax.ShapeDtypeStruct((1, 1), jnp.float32),
    )(grouped_c, grouped_r, probs.reshape(F_pad, 1))

    return out[0, 0]


# speedup vs baseline: 17.4717x; 1.2366x over previous
import functools

import jax
import jax.numpy as jnp
from jax.experimental import pallas as pl
from jax.experimental.pallas import tpu as pltpu

_TE = 512
_TM = 1024
_TN = 1024

_T1 = (1.0 + 1e-6) ** 2
_T2 = (1e-5 - 1e-8) ** 2


def _feature_body(vpad_ref, vpadT_ref, ac_ref, bc_ref, ar_ref, br_ref,
                  rowV_ref, rowC_ref, colV_ref, colC_ref, *, n_edges, V):
    t = pl.program_id(0)
    te = ac_ref.shape[0]

    lane_iota = jax.lax.broadcasted_iota(jnp.int32, (te, V), 1)
    oh_a = (lane_iota == ac_ref[...]).astype(jnp.float32)
    oh_b = (lane_iota == bc_ref[...]).astype(jnp.float32)
    pa = jnp.dot(oh_a, vpad_ref[...], preferred_element_type=jnp.float32)
    pb = jnp.dot(oh_b, vpad_ref[...], preferred_element_type=jnp.float32)
    v = pb - pa + 1e-8
    c = (pa + pb) * 0.5
    n = (v[:, 0:1] * v[:, 0:1] + v[:, 1:2] * v[:, 1:2]
         + v[:, 2:3] * v[:, 2:3])
    p = ((c[:, 0:1] * c[:, 0:1] + c[:, 1:2] * c[:, 1:2]
          + c[:, 2:3] * c[:, 2:3]) * 0.5 - _T1 * 0.25)
    ge = t * te + jax.lax.broadcasted_iota(jnp.int32, (te, 1), 0)
    p = jnp.where(ge < n_edges, p, 1e30)
    rowV_ref[...] = jnp.concatenate([v[:, 0:3], n], axis=1)
    rowC_ref[...] = jnp.concatenate([c[:, 0:3], p], axis=1)

    sub_iota = jax.lax.broadcasted_iota(jnp.int32, (V, te), 0)
    oh_at = (sub_iota == ar_ref[...]).astype(jnp.float32)
    oh_bt = (sub_iota == br_ref[...]).astype(jnp.float32)
    pat = jnp.dot(vpadT_ref[...], oh_at, preferred_element_type=jnp.float32)
    pbt = jnp.dot(vpadT_ref[...], oh_bt, preferred_element_type=jnp.float32)
    vt = pbt - pat + 1e-8
    ct = (pat + pbt) * 0.5
    nt = (vt[0:1, :] * vt[0:1, :] + vt[1:2, :] * vt[1:2, :]
          + vt[2:3, :] * vt[2:3, :])
    pt = ((ct[0:1, :] * ct[0:1, :] + ct[1:2, :] * ct[1:2, :]
           + ct[2:3, :] * ct[2:3, :]) * 0.5 - _T1 * 0.25)
    ger = t * te + jax.lax.broadcasted_iota(jnp.int32, (1, te), 1)
    pt = jnp.where(ger < n_edges, pt, 1e30)
    colV_ref[...] = jnp.concatenate([vt[0:3, :], nt], axis=0)
    colC_ref[...] = jnp.concatenate([ct[0:3, :], pt], axis=0)


def _pair_body(rowV_ref, rowC_ref, colV_ref, colC_ref, degc_ref, degr_ref,
               racc_ref):
    i = pl.program_id(0)
    j = pl.program_id(1)
    ni = pl.num_programs(0)
    nj = pl.num_programs(1)

    @pl.when((i == 0) & (j == 0))
    def _init():
        degc_ref[...] = jnp.zeros_like(degc_ref)
        racc_ref[...] = jnp.zeros_like(racc_ref)

    def predicate():
        vix = rowV_ref[:, 0:1]
        viy = rowV_ref[:, 1:2]
        viz = rowV_ref[:, 2:3]
        ni = rowV_ref[:, 3:4]
        vjx = colV_ref[0:1, :]
        vjy = colV_ref[1:2, :]
        vjz = colV_ref[2:3, :]
        nj = colV_ref[3:4, :]

        g = (rowC_ref[:, 0:1] * colC_ref[0:1, :]
             + rowC_ref[:, 1:2] * colC_ref[1:2, :]
             + rowC_ref[:, 2:3] * colC_ref[2:3, :])
        s = vix * vjx + viy * vjy + viz * vjz
        h = ni * nj - s * s
        return (g > rowC_ref[:, 3:4] + colC_ref[3:4, :]) & (h > _T2)

    def accumulate(contrib):
        degc_ref[0:1, pl.ds(j * _TN, _TN)] += jnp.sum(
            contrib, axis=0, keepdims=True)
        part = sum(contrib[:, k * 128:(k + 1) * 128] for k in range(1, _TN // 128)) + contrib[:, 0:128]
        racc_ref[pl.ds(i * _TM, _TM), :] += part

    @pl.when(j > i)
    def _upper():
        accumulate(predicate().astype(jnp.float32))

    @pl.when(j == i)
    def _diag():
        li = jax.lax.broadcasted_iota(jnp.int32, (_TM, 1), 0)
        lj = jax.lax.broadcasted_iota(jnp.int32, (1, _TN), 1)
        accumulate((predicate() & (li < lj)).astype(jnp.float32))

    @pl.when((i == ni - 1) & (j == nj - 1))
    def _flush():
        degr_ref[...] = jnp.sum(racc_ref[...], axis=1, keepdims=True)


def _combine_body(x_ref, y_ref, p_ref, out_ref, *, inv_f):
    cc = jnp.sum(x_ref[...] + y_ref[...], axis=1, keepdims=True)
    cc = jnp.clip(cc, 0.0, 100.0)
    out_ref[...] = jnp.sum(cc * p_ref[...], keepdims=True) * inv_f


def kernel(vertices, faces, face_probs):
    V = vertices.shape[0]
    F = faces.shape[0]
    E = 3 * F
    E_pad = ((E + _TM - 1) // _TM) * _TM
    F_pad = ((F + 7) // 8) * 8

    a = jnp.concatenate([faces[:, 0], faces[:, 1], faces[:, 2]])
    b = jnp.concatenate([faces[:, 1], faces[:, 2], faces[:, 0]])
    a = jnp.pad(a, (0, E_pad - E))
    b = jnp.pad(b, (0, E_pad - E))

    vpad = jnp.pad(vertices.astype(jnp.float32), ((0, 0), (0, 5)))
    vpadT = vpad.T

    feat = pl.pallas_call(
        functools.partial(_feature_body, n_edges=E, V=V),
        grid=(E_pad // _TE,),
        in_specs=[
            pl.BlockSpec((V, 8), lambda t: (0, 0)),
            pl.BlockSpec((8, V), lambda t: (0, 0)),
            pl.BlockSpec((_TE, 1), lambda t: (t, 0)),
            pl.BlockSpec((_TE, 1), lambda t: (t, 0)),
            pl.BlockSpec((1, _TE), lambda t: (0, t)),
            pl.BlockSpec((1, _TE), lambda t: (0, t)),
        ],
        out_specs=[
            pl.BlockSpec((_TE, 4), lambda t: (t, 0)),
            pl.BlockSpec((_TE, 4), lambda t: (t, 0)),
            pl.BlockSpec((4, _TE), lambda t: (0, t)),
            pl.BlockSpec((4, _TE), lambda t: (0, t)),
        ],
        out_shape=[
            jax.ShapeDtypeStruct((E_pad, 4), jnp.float32),
            jax.ShapeDtypeStruct((E_pad, 4), jnp.float32),
            jax.ShapeDtypeStruct((4, E_pad), jnp.float32),
            jax.ShapeDtypeStruct((4, E_pad), jnp.float32),
        ],
    )(vpad, vpadT, a.reshape(E_pad, 1), b.reshape(E_pad, 1),
      a.reshape(1, E_pad), b.reshape(1, E_pad))
    rowV, rowC, colV, colC = feat

    degc, degr = pl.pallas_call(
        _pair_body,
        grid=(E_pad // _TM, E_pad // _TN),
        in_specs=[
            pl.BlockSpec((_TM, 4), lambda i, j: (i, 0)),
            pl.BlockSpec((_TM, 4), lambda i, j: (i, 0)),
            pl.BlockSpec((4, _TN), lambda i, j: (0, j)),
            pl.BlockSpec((4, _TN), lambda i, j: (0, j)),
        ],
        out_specs=[
            pl.BlockSpec((8, E_pad), lambda i, j: (0, 0)),
            pl.BlockSpec((E_pad, 1), lambda i, j: (0, 0)),
        ],
        out_shape=[
            jax.ShapeDtypeStruct((8, E_pad), jnp.float32),
            jax.ShapeDtypeStruct((E_pad, 1), jnp.float32),
        ],
        scratch_shapes=[pltpu.VMEM((E_pad, 128), jnp.float32)],
    )(rowV, rowC, colV, colC)

    grouped_c = jnp.pad(degc[0, :E].reshape(F, 3), ((0, F_pad - F), (0, 0)))
    grouped_r = jnp.pad(degr[:E, 0].reshape(F, 3), ((0, F_pad - F), (0, 0)))
    probs = jnp.pad(face_probs.astype(jnp.float32), (0, F_pad - F))

    out = pl.pallas_call(
        functools.partial(_combine_body, inv_f=1.0 / F),
        in_specs=[
            pl.BlockSpec((F_pad, 3), lambda: (0, 0)),
            pl.BlockSpec((F_pad, 3), lambda: (0, 0)),
            pl.BlockSpec((F_pad, 1), lambda: (0, 0)),
        ],
        out_specs=pl.BlockSpec((1, 1), lambda: (0, 0)),
        out_shape=jax.ShapeDtypeStruct((1, 1), jnp.float32),
    )(grouped_c, grouped_r, probs.reshape(F_pad, 1))

    return out[0, 0]


# speedup vs baseline: 17.9333x; 1.0264x over previous
import functools

import jax
import jax.numpy as jnp
from jax import lax
from jax.experimental import pallas as pl
from jax.experimental.pallas import tpu as pltpu
from jax.experimental.pallas import tpu_sc as plsc

_TM = 1024
_TN = 1024
_NW = 32

_T1 = (1.0 + 1e-6) ** 2
_T2 = (1e-5 - 1e-8) ** 2


def _sc_gather_call(table, aidx, bidx, e_pad):
    bpw = e_pad // _NW
    mesh = plsc.VectorSubcoreMesh(core_axis_name="c", subcore_axis_name="s")

    def body(table_hbm, aidx_hbm, bidx_hbm, pa_hbm, pb_hbm,
             aidx_v, rows_a, bidx_v, rows_b, sem_a, sem_b):
        wid = lax.axis_index("s") * 2 + lax.axis_index("c")
        base = wid * bpw
        pltpu.sync_copy(aidx_hbm.at[pl.ds(base, bpw)], aidx_v)
        pltpu.sync_copy(bidx_hbm.at[pl.ds(base, bpw)], bidx_v)
        ca = pltpu.async_copy(table_hbm.at[aidx_v], rows_a, sem_a)
        cb = pltpu.async_copy(table_hbm.at[bidx_v], rows_b, sem_b)
        ca.wait()
        cb.wait()
        pltpu.sync_copy(rows_a, pa_hbm.at[pl.ds(base, bpw)])
        pltpu.sync_copy(rows_b, pb_hbm.at[pl.ds(base, bpw)])

    k = functools.partial(
        pl.kernel, mesh=mesh,
        compiler_params=pltpu.CompilerParams(use_tc_tiling_on_sc=False),
        out_type=[jax.ShapeDtypeStruct((e_pad, 16), jnp.float32),
                  jax.ShapeDtypeStruct((e_pad, 16), jnp.float32)],
        scratch_types=[
            pltpu.VMEM((bpw,), jnp.int32),
            pltpu.VMEM((bpw, 16), jnp.float32),
            pltpu.VMEM((bpw,), jnp.int32),
            pltpu.VMEM((bpw, 16), jnp.float32),
            pltpu.SemaphoreType.DMA,
            pltpu.SemaphoreType.DMA,
        ],
    )(body)
    return k(table, aidx, bidx)


def _pair_body(pa_ref, pb_ref, degc_ref, degr_ref,
               rowV_ref, rowC_ref, colV_ref, colC_ref, racc_ref, *, n_edges):
    i = pl.program_id(0)
    j = pl.program_id(1)
    ni = pl.num_programs(0)
    nj = pl.num_programs(1)

    @pl.when((i == 0) & (j == 0))
    def _prologue():
        degc_ref[...] = jnp.zeros_like(degc_ref)
        racc_ref[...] = jnp.zeros_like(racc_ref)
        vx = pb_ref[:, 0:1] - pa_ref[:, 0:1] + 1e-8
        vy = pb_ref[:, 1:2] - pa_ref[:, 1:2] + 1e-8
        vz = pb_ref[:, 2:3] - pa_ref[:, 2:3] + 1e-8
        cx = (pa_ref[:, 0:1] + pb_ref[:, 0:1]) * 0.5
        cy = (pa_ref[:, 1:2] + pb_ref[:, 1:2]) * 0.5
        cz = (pa_ref[:, 2:3] + pb_ref[:, 2:3]) * 0.5
        n = vx * vx + vy * vy + vz * vz
        p = (cx * cx + cy * cy + cz * cz) * 0.5 - _T1 * 0.25
        e_pad = pa_ref.shape[0]
        ge = jax.lax.broadcasted_iota(jnp.int32, (e_pad, 1), 0)
        p = jnp.where(ge < n_edges, p, 1e30)
        rowv = jnp.concatenate([vx, vy, vz, n], axis=1)
        rowc = jnp.concatenate([cx, cy, cz, p], axis=1)
        rowV_ref[...] = rowv
        rowC_ref[...] = rowc
        colV_ref[...] = jnp.transpose(rowv, (1, 0))
        colC_ref[...] = jnp.transpose(rowc, (1, 0))

    def predicate():
        rv = rowV_ref[pl.ds(i * _TM, _TM), :]
        rc = rowC_ref[pl.ds(i * _TM, _TM), :]
        cv = colV_ref[:, pl.ds(j * _TN, _TN)]
        cc = colC_ref[:, pl.ds(j * _TN, _TN)]
        g = (rc[:, 0:1] * cc[0:1, :]
             + rc[:, 1:2] * cc[1:2, :]
             + rc[:, 2:3] * cc[2:3, :])
        s = (rv[:, 0:1] * cv[0:1, :]
             + rv[:, 1:2] * cv[1:2, :]
             + rv[:, 2:3] * cv[2:3, :])
        h = rv[:, 3:4] * cv[3:4, :] - s * s
        return (g > rc[:, 3:4] + cc[3:4, :]) & (h > _T2)

    def accumulate(contrib):
        degc_ref[0:1, pl.ds(j * _TN, _TN)] += jnp.sum(
            contrib, axis=0, keepdims=True)
        part = (sum(contrib[:, k * 128:(k + 1) * 128]
                    for k in range(1, _TN // 128))
                + contrib[:, 0:128])
        racc_ref[pl.ds(i * _TM, _TM), :] += part

    @pl.when(j > i)
    def _upper():
        accumulate(predicate().astype(jnp.float32))

    @pl.when(j == i)
    def _diag():
        li = jax.lax.broadcasted_iota(jnp.int32, (_TM, 1), 0)
        lj = jax.lax.broadcasted_iota(jnp.int32, (1, _TN), 1)
        accumulate((predicate() & (li < lj)).astype(jnp.float32))

    @pl.when((i == ni - 1) & (j == nj - 1))
    def _flush():
        degr_ref[...] = jnp.sum(racc_ref[...], axis=1, keepdims=True)


def _combine_body(x_ref, y_ref, p_ref, out_ref, *, inv_f):
    cc = jnp.sum(x_ref[...] + y_ref[...], axis=1, keepdims=True)
    cc = jnp.clip(cc, 0.0, 100.0)
    out_ref[...] = jnp.sum(cc * p_ref[...], keepdims=True) * inv_f


def kernel(vertices, faces, face_probs):
    F = faces.shape[0]
    E = 3 * F
    E_pad = ((E + _TM - 1) // _TM) * _TM
    F_pad = ((F + 7) // 8) * 8

    a = jnp.concatenate([faces[:, 0], faces[:, 1], faces[:, 2]])
    b = jnp.concatenate([faces[:, 1], faces[:, 2], faces[:, 0]])
    a = jnp.pad(a, (0, E_pad - E))
    b = jnp.pad(b, (0, E_pad - E))

    table = jnp.pad(vertices.astype(jnp.float32), ((0, 0), (0, 13)))
    pa, pb = _sc_gather_call(table, a, b, E_pad)

    degc, degr = pl.pallas_call(
        functools.partial(_pair_body, n_edges=E),
        grid=(E_pad // _TM, E_pad // _TN),
        in_specs=[
            pl.BlockSpec((E_pad, 16), lambda i, j: (0, 0)),
            pl.BlockSpec((E_pad, 16), lambda i, j: (0, 0)),
        ],
        out_specs=[
            pl.BlockSpec((8, E_pad), lambda i, j: (0, 0)),
            pl.BlockSpec((E_pad, 1), lambda i, j: (0, 0)),
        ],
        out_shape=[
            jax.ShapeDtypeStruct((8, E_pad), jnp.float32),
            jax.ShapeDtypeStruct((E_pad, 1), jnp.float32),
        ],
        scratch_shapes=[
            pltpu.VMEM((E_pad, 4), jnp.float32),
            pltpu.VMEM((E_pad, 4), jnp.float32),
            pltpu.VMEM((4, E_pad), jnp.float32),
            pltpu.VMEM((4, E_pad), jnp.float32),
            pltpu.VMEM((E_pad, 128), jnp.float32),
        ],
    )(pa, pb)

    grouped_c = jnp.pad(degc[0, :E].reshape(F, 3), ((0, F_pad - F), (0, 0)))
    grouped_r = jnp.pad(degr[:E, 0].reshape(F, 3), ((0, F_pad - F), (0, 0)))
    probs = jnp.pad(face_probs.astype(jnp.float32), (0, F_pad - F))

    out = pl.pallas_call(
        functools.partial(_combine_body, inv_f=1.0 / F),
        in_specs=[
            pl.BlockSpec((F_pad, 3), lambda: (0, 0)),
            pl.BlockSpec((F_pad, 3), lambda: (0, 0)),
            pl.BlockSpec((F_pad, 1), lambda: (0, 0)),
        ],
        out_specs=pl.BlockSpec((1, 1), lambda: (0, 0)),
        out_shape=jax.ShapeDtypeStruct((1, 1), jnp.float32),
    )(grouped_c, grouped_r, probs.reshape(F_pad, 1))

    return out[0, 0]


# speedup vs baseline: 19.7184x; 1.0995x over previous
import functools

import jax
import jax.numpy as jnp
from jax import lax
from jax.experimental import pallas as pl
from jax.experimental.pallas import tpu as pltpu
from jax.experimental.pallas import tpu_sc as plsc

_TM = 1024
_TN = 1024
_NW = 32

_T1 = (1.0 + 1e-6) ** 2
_T2 = (1e-5 - 1e-8) ** 2


def _sc_gather_call(table, aidx, bidx, e_pad):
    bpw = e_pad // _NW
    mesh = plsc.VectorSubcoreMesh(core_axis_name="c", subcore_axis_name="s")

    def body(table_hbm, aidx_hbm, bidx_hbm, pa_hbm, pb_hbm,
             aidx_v, rows_a, bidx_v, rows_b, sem_a, sem_b):
        wid = lax.axis_index("s") * 2 + lax.axis_index("c")
        base = wid * bpw
        pltpu.sync_copy(aidx_hbm.at[pl.ds(base, bpw)], aidx_v)
        pltpu.sync_copy(bidx_hbm.at[pl.ds(base, bpw)], bidx_v)
        ca = pltpu.async_copy(table_hbm.at[aidx_v], rows_a, sem_a)
        cb = pltpu.async_copy(table_hbm.at[bidx_v], rows_b, sem_b)
        ca.wait()
        cb.wait()
        pltpu.sync_copy(rows_a, pa_hbm.at[pl.ds(base, bpw)])
        pltpu.sync_copy(rows_b, pb_hbm.at[pl.ds(base, bpw)])

    k = functools.partial(
        pl.kernel, mesh=mesh,
        compiler_params=pltpu.CompilerParams(use_tc_tiling_on_sc=False),
        out_type=[jax.ShapeDtypeStruct((e_pad, 16), jnp.float32),
                  jax.ShapeDtypeStruct((e_pad, 16), jnp.float32)],
        scratch_types=[
            pltpu.VMEM((bpw,), jnp.int32),
            pltpu.VMEM((bpw, 16), jnp.float32),
            pltpu.VMEM((bpw,), jnp.int32),
            pltpu.VMEM((bpw, 16), jnp.float32),
            pltpu.SemaphoreType.DMA,
            pltpu.SemaphoreType.DMA,
        ],
    )(body)
    return k(table, aidx, bidx)


def _pair_body(pa_ref, pb_ref, pexp_ref, out_ref,
               rowV_ref, rowC_ref, colV_ref, colC_ref, racc_ref, degc_ref,
               *, n_edges, inv_f):
    i = pl.program_id(0)
    j = pl.program_id(1)
    ni = pl.num_programs(0)
    nj = pl.num_programs(1)

    @pl.when((i == 0) & (j == 0))
    def _prologue():
        degc_ref[...] = jnp.zeros_like(degc_ref)
        racc_ref[...] = jnp.zeros_like(racc_ref)
        vx = pb_ref[:, 0:1] - pa_ref[:, 0:1] + 1e-8
        vy = pb_ref[:, 1:2] - pa_ref[:, 1:2] + 1e-8
        vz = pb_ref[:, 2:3] - pa_ref[:, 2:3] + 1e-8
        cx = (pa_ref[:, 0:1] + pb_ref[:, 0:1]) * 0.5
        cy = (pa_ref[:, 1:2] + pb_ref[:, 1:2]) * 0.5
        cz = (pa_ref[:, 2:3] + pb_ref[:, 2:3]) * 0.5
        n = vx * vx + vy * vy + vz * vz
        p = (cx * cx + cy * cy + cz * cz) * 0.5 - _T1 * 0.25
        e_pad = pa_ref.shape[0]
        ge = jax.lax.broadcasted_iota(jnp.int32, (e_pad, 1), 0)
        p = jnp.where(ge < n_edges, p, 1e30)
        rowv = jnp.concatenate([vx, vy, vz, n], axis=1)
        rowc = jnp.concatenate([cx, cy, cz, p], axis=1)
        rowV_ref[...] = rowv
        rowC_ref[...] = rowc
        colV_ref[...] = jnp.transpose(rowv, (1, 0))
        colC_ref[...] = jnp.transpose(rowc, (1, 0))

    def predicate():
        rv = rowV_ref[pl.ds(i * _TM, _TM), :]
        rc = rowC_ref[pl.ds(i * _TM, _TM), :]
        cv = colV_ref[:, pl.ds(j * _TN, _TN)]
        cc = colC_ref[:, pl.ds(j * _TN, _TN)]
        g = (rc[:, 0:1] * cc[0:1, :]
             + rc[:, 1:2] * cc[1:2, :]
             + rc[:, 2:3] * cc[2:3, :])
        s = (rv[:, 0:1] * cv[0:1, :]
             + rv[:, 1:2] * cv[1:2, :]
             + rv[:, 2:3] * cv[2:3, :])
        h = rv[:, 3:4] * cv[3:4, :] - s * s
        return (g > rc[:, 3:4] + cc[3:4, :]) & (h > _T2)

    def accumulate(contrib):
        degc_ref[0:1, pl.ds(j * _TN, _TN)] += jnp.sum(
            contrib, axis=0, keepdims=True)
        part = (sum(contrib[:, k * 128:(k + 1) * 128]
                    for k in range(1, _TN // 128))
                + contrib[:, 0:128])
        racc_ref[pl.ds(i * _TM, _TM), :] += part

    @pl.when(j > i)
    def _upper():
        accumulate(predicate().astype(jnp.float32))

    @pl.when(j == i)
    def _diag():
        li = jax.lax.broadcasted_iota(jnp.int32, (_TM, 1), 0)
        lj = jax.lax.broadcasted_iota(jnp.int32, (1, _TN), 1)
        accumulate((predicate() & (li < lj)).astype(jnp.float32))

    @pl.when((i == ni - 1) & (j == nj - 1))
    def _flush():
        degr = jnp.transpose(
            jnp.sum(racc_ref[...], axis=1, keepdims=True), (1, 0))
        tot = degc_ref[0:1, :] + degr
        z1 = jnp.zeros((1, 1), jnp.float32)
        z2 = jnp.zeros((1, 2), jnp.float32)
        sh1 = jnp.concatenate([tot[:, 1:], z1], axis=1)
        sh2 = jnp.concatenate([tot[:, 2:], z2], axis=1)
        cc = jnp.clip(tot + sh1 + sh2, 0.0, 100.0)
        out_ref[...] = jnp.sum(cc * pexp_ref[...], keepdims=True) * inv_f


def kernel(vertices, faces, face_probs):
    F = faces.shape[0]
    E = 3 * F
    E_pad = ((E + _TM - 1) // _TM) * _TM

    a = jnp.concatenate([faces[:, 0], faces[:, 1], faces[:, 2]])
    b = jnp.concatenate([faces[:, 1], faces[:, 2], faces[:, 0]])
    a = jnp.pad(a, (0, E_pad - E))
    b = jnp.pad(b, (0, E_pad - E))

    table = jnp.pad(vertices.astype(jnp.float32), ((0, 0), (0, 13)))
    pa, pb = _sc_gather_call(table, a, b, E_pad)

    pexp = jnp.pad(
        jnp.pad(face_probs.astype(jnp.float32).reshape(F, 1),
                ((0, 0), (0, 2))).reshape(E),
        (0, E_pad - E)).reshape(1, E_pad)

    out = pl.pallas_call(
        functools.partial(_pair_body, n_edges=E, inv_f=1.0 / F),
        grid=(E_pad // _TM, E_pad // _TN),
        in_specs=[
            pl.BlockSpec((E_pad, 16), lambda i, j: (0, 0)),
            pl.BlockSpec((E_pad, 16), lambda i, j: (0, 0)),
            pl.BlockSpec((1, E_pad), lambda i, j: (0, 0)),
        ],
        out_specs=pl.BlockSpec((1, 1), lambda i, j: (0, 0)),
        out_shape=jax.ShapeDtypeStruct((1, 1), jnp.float32),
        scratch_shapes=[
            pltpu.VMEM((E_pad, 4), jnp.float32),
            pltpu.VMEM((E_pad, 4), jnp.float32),
            pltpu.VMEM((4, E_pad), jnp.float32),
            pltpu.VMEM((4, E_pad), jnp.float32),
            pltpu.VMEM((E_pad, 128), jnp.float32),
            pltpu.VMEM((8, E_pad), jnp.float32),
        ],
    )(pa, pb, pexp)

    return out[0, 0]


# speedup vs baseline: 19.7932x; 1.0038x over previous
import functools

import jax
import jax.numpy as jnp
from jax import lax
from jax.experimental import pallas as pl
from jax.experimental.pallas import tpu as pltpu
from jax.experimental.pallas import tpu_sc as plsc

_TM = 1024
_TN = 1024
_NW = 32

_T1 = (1.0 + 1e-6) ** 2
_T2 = (1e-5 - 1e-8) ** 2


def _sc_gather_call(table, aidx, bidx, e_pad):
    bpw = e_pad // _NW
    mesh = plsc.VectorSubcoreMesh(core_axis_name="c", subcore_axis_name="s")

    def body(table_hbm, aidx_hbm, bidx_hbm, pa_hbm, pb_hbm,
             aidx_v, rows_a, bidx_v, rows_b, sem_a, sem_b):
        wid = lax.axis_index("s") * 2 + lax.axis_index("c")
        base = wid * bpw
        pltpu.sync_copy(aidx_hbm.at[pl.ds(base, bpw)], aidx_v)
        pltpu.sync_copy(bidx_hbm.at[pl.ds(base, bpw)], bidx_v)
        ca = pltpu.async_copy(table_hbm.at[aidx_v], rows_a, sem_a)
        cb = pltpu.async_copy(table_hbm.at[bidx_v], rows_b, sem_b)
        ca.wait()
        cb.wait()
        pltpu.sync_copy(rows_a, pa_hbm.at[pl.ds(base, bpw)])
        pltpu.sync_copy(rows_b, pb_hbm.at[pl.ds(base, bpw)])

    k = functools.partial(
        pl.kernel, mesh=mesh,
        compiler_params=pltpu.CompilerParams(use_tc_tiling_on_sc=False),
        out_type=[jax.ShapeDtypeStruct((e_pad, 16), jnp.float32),
                  jax.ShapeDtypeStruct((e_pad, 16), jnp.float32)],
        scratch_types=[
            pltpu.VMEM((bpw,), jnp.int32),
            pltpu.VMEM((bpw, 16), jnp.float32),
            pltpu.VMEM((bpw,), jnp.int32),
            pltpu.VMEM((bpw, 16), jnp.float32),
            pltpu.SemaphoreType.DMA,
            pltpu.SemaphoreType.DMA,
        ],
    )(body)
    return k(table, aidx, bidx)


def _pair_body(pa_ref, pb_ref, pexp_ref, out_ref,
               rowV_ref, rowC_ref, colV_ref, colC_ref, racc_ref, degc_ref,
               *, n_edges, inv_f, nb):
    t = pl.program_id(0)
    nt = pl.num_programs(0)
    row_starts = [nb * r - (r * (r - 1)) // 2 for r in range(1, nb)]
    i = sum(((t >= o).astype(jnp.int32) for o in row_starts),
            jnp.zeros((), jnp.int32))
    j = t - (nb * i - (i * (i - 1)) // 2) + i

    @pl.when(t == 0)
    def _prologue():
        degc_ref[...] = jnp.zeros_like(degc_ref)
        racc_ref[...] = jnp.zeros_like(racc_ref)
        vx = pb_ref[:, 0:1] - pa_ref[:, 0:1] + 1e-8
        vy = pb_ref[:, 1:2] - pa_ref[:, 1:2] + 1e-8
        vz = pb_ref[:, 2:3] - pa_ref[:, 2:3] + 1e-8
        cx = (pa_ref[:, 0:1] + pb_ref[:, 0:1]) * 0.5
        cy = (pa_ref[:, 1:2] + pb_ref[:, 1:2]) * 0.5
        cz = (pa_ref[:, 2:3] + pb_ref[:, 2:3]) * 0.5
        n = vx * vx + vy * vy + vz * vz
        p = (cx * cx + cy * cy + cz * cz) * 0.5 - _T1 * 0.25
        e_pad = pa_ref.shape[0]
        ge = jax.lax.broadcasted_iota(jnp.int32, (e_pad, 1), 0)
        p = jnp.where(ge < n_edges, p, 1e30)
        rowv = jnp.concatenate([vx, vy, vz, n], axis=1)
        rowc = jnp.concatenate([cx, cy, cz, p], axis=1)
        rowV_ref[...] = rowv
        rowC_ref[...] = rowc
        colV_ref[...] = jnp.transpose(rowv, (1, 0))
        colC_ref[...] = jnp.transpose(rowc, (1, 0))

    def predicate():
        rv = rowV_ref[pl.ds(i * _TM, _TM), :]
        rc = rowC_ref[pl.ds(i * _TM, _TM), :]
        cv = colV_ref[:, pl.ds(j * _TN, _TN)]
        cc = colC_ref[:, pl.ds(j * _TN, _TN)]
        g = (rc[:, 0:1] * cc[0:1, :]
             + rc[:, 1:2] * cc[1:2, :]
             + rc[:, 2:3] * cc[2:3, :])
        s = (rv[:, 0:1] * cv[0:1, :]
             + rv[:, 1:2] * cv[1:2, :]
             + rv[:, 2:3] * cv[2:3, :])
        h = rv[:, 3:4] * cv[3:4, :] - s * s
        return (g > rc[:, 3:4] + cc[3:4, :]) & (h > _T2)

    def accumulate(contrib):
        degc_ref[0:1, pl.ds(j * _TN, _TN)] += jnp.sum(
            contrib, axis=0, keepdims=True)
        part = (sum(contrib[:, k * 128:(k + 1) * 128]
                    for k in range(1, _TN // 128))
                + contrib[:, 0:128])
        racc_ref[pl.ds(i * _TM, _TM), :] += part

    @pl.when(j > i)
    def _upper():
        accumulate(predicate().astype(jnp.float32))

    @pl.when(j == i)
    def _diag():
        li = jax.lax.broadcasted_iota(jnp.int32, (_TM, 1), 0)
        lj = jax.lax.broadcasted_iota(jnp.int32, (1, _TN), 1)
        accumulate((predicate() & (li < lj)).astype(jnp.float32))

    @pl.when(t == nt - 1)
    def _flush():
        degr = jnp.transpose(
            jnp.sum(racc_ref[...], axis=1, keepdims=True), (1, 0))
        tot = degc_ref[0:1, :] + degr
        z1 = jnp.zeros((1, 1), jnp.float32)
        z2 = jnp.zeros((1, 2), jnp.float32)
        sh1 = jnp.concatenate([tot[:, 1:], z1], axis=1)
        sh2 = jnp.concatenate([tot[:, 2:], z2], axis=1)
        cc = jnp.clip(tot + sh1 + sh2, 0.0, 100.0)
        out_ref[...] = jnp.sum(cc * pexp_ref[...], keepdims=True) * inv_f


def kernel(vertices, faces, face_probs):
    F = faces.shape[0]
    E = 3 * F
    E_pad = ((E + _TM - 1) // _TM) * _TM

    a = jnp.concatenate([faces[:, 0], faces[:, 1], faces[:, 2]])
    b = jnp.concatenate([faces[:, 1], faces[:, 2], faces[:, 0]])
    a = jnp.pad(a, (0, E_pad - E))
    b = jnp.pad(b, (0, E_pad - E))

    table = jnp.pad(vertices.astype(jnp.float32), ((0, 0), (0, 13)))
    pa, pb = _sc_gather_call(table, a, b, E_pad)

    pexp = jnp.pad(
        jnp.pad(face_probs.astype(jnp.float32).reshape(F, 1),
                ((0, 0), (0, 2))).reshape(E),
        (0, E_pad - E)).reshape(1, E_pad)

    nb = E_pad // _TM
    out = pl.pallas_call(
        functools.partial(_pair_body, n_edges=E, inv_f=1.0 / F, nb=nb),
        grid=(nb * (nb + 1) // 2,),
        in_specs=[
            pl.BlockSpec((E_pad, 16), lambda t: (0, 0)),
            pl.BlockSpec((E_pad, 16), lambda t: (0, 0)),
            pl.BlockSpec((1, E_pad), lambda t: (0, 0)),
        ],
        out_specs=pl.BlockSpec((1, 1), lambda t: (0, 0)),
        out_shape=jax.ShapeDtypeStruct((1, 1), jnp.float32),
        scratch_shapes=[
            pltpu.VMEM((E_pad, 4), jnp.float32),
            pltpu.VMEM((E_pad, 4), jnp.float32),
            pltpu.VMEM((4, E_pad), jnp.float32),
            pltpu.VMEM((4, E_pad), jnp.float32),
            pltpu.VMEM((E_pad, 128), jnp.float32),
            pltpu.VMEM((8, E_pad), jnp.float32),
        ],
    )(pa, pb, pexp)

    return out[0, 0]
